# baseline (speedup 1.0000x reference)
import functools

import jax
import jax.numpy as jnp
from jax import lax
from jax.experimental import pallas as pl
from jax.experimental.pallas import tpu as pltpu
from jax.experimental.pallas import tpu_sc as plsc

N = 10000
E = 320000
C = 128
NUM_LAYERS = 3
NUM_GRAPHS = 64

NC = 2
NS = 16
NW = NC * NS
EPW = E // NW
CHUNK = 104
NCHUNK = -(-EPW // CHUNK)
EPWP = NCHUNK * CHUNK
NRING = 2
NQUAD = NCHUNK // NRING
NA = N + 8
RPT = 624
ZROWS = 16

ROWS_BLK = 1000
NBLK = N // ROWS_BLK


def _sc_scatter_kernel(m_hbm, src_hbm, dst_hbm, out_hbm,
                       sidx, didx, rows, acc, gsems, ssems):
    c = lax.axis_index("c")
    s = lax.axis_index("s")
    wid = c * NS + s

    zero16 = jnp.zeros((16,), jnp.float32)

    def zero_row(r, carry):
        for lane in range(C // 16):
            rows[0][r, pl.ds(lane * 16, 16)] = zero16
        return carry

    lax.fori_loop(0, ZROWS, zero_row, 0)
    zsrc = rows[0].at[pl.ds(0, ZROWS)]

    def zero_acc(t, carry):
        pltpu.sync_copy(zsrc, acc.at[pl.ds(s * RPT + t * ZROWS, ZROWS)])
        return carry

    lax.fori_loop(0, RPT // ZROWS, zero_acc, 0)

    @pl.when(s == NS - 1)
    def _():
        pltpu.sync_copy(zsrc, acc.at[pl.ds(NS * RPT, ZROWS)])
        pltpu.sync_copy(rows[0].at[pl.ds(0, NA - NS * RPT - ZROWS)],
                        acc.at[pl.ds(NS * RPT + ZROWS, NA - NS * RPT - ZROWS)])

    plsc.subcore_barrier()

    pltpu.sync_copy(src_hbm.at[pl.ds(wid * EPWP, EPWP)], sidx)
    pltpu.sync_copy(dst_hbm.at[wid], didx)

    def start_gather(j, b):
        pltpu.async_copy(m_hbm.at[sidx.at[pl.ds(j * CHUNK, CHUNK)]],
                         rows[b], gsems[b])

    def wait_gather(b):
        pltpu.make_async_copy(m_hbm.at[pl.ds(0, CHUNK)],
                              rows[b], gsems[b]).wait()

    for b in range(NRING):
        start_gather(b, b)

    def ring_round(jq, carry):
        j0 = jq * NRING
        descs = []
        for b in range(NRING):
            wait_gather(b)
            descs.append(pltpu.async_copy(rows[b], acc.at[didx.at[j0 + b]],
                                          ssems[b], add=True))
        for b in range(NRING):
            j4 = j0 + NRING + b
            descs[b].wait()

            @pl.when(j4 < NCHUNK)
            def _():
                start_gather(j4, b)
        return carry

    lax.fori_loop(0, NQUAD, ring_round, 0)
    for b in range(NCHUNK - NQUAD * NRING):
        wait_gather(b)
        pltpu.sync_copy(rows[b], acc.at[didx.at[NQUAD * NRING + b]], add=True)
    plsc.subcore_barrier()

    pltpu.sync_copy(acc.at[pl.ds(s * RPT, RPT)],
                    out_hbm.at[pl.ds(c * N + s * RPT, RPT)])

    @pl.when(s == NS - 1)
    def _():
        pltpu.sync_copy(acc.at[pl.ds(NS * RPT, N - NS * RPT)],
                        out_hbm.at[pl.ds(c * N + NS * RPT, N - NS * RPT)])


def _sc_scatter(m, srcp, dst3):
    mesh = plsc.VectorSubcoreMesh(core_axis_name="c", subcore_axis_name="s")
    return pl.kernel(
        _sc_scatter_kernel,
        out_type=jax.ShapeDtypeStruct((NC * N, C), jnp.float32),
        mesh=mesh,
        scratch_types=[
            pltpu.VMEM((EPWP,), jnp.int32),
            pltpu.VMEM((NCHUNK, CHUNK), jnp.int32),
            [pltpu.VMEM((CHUNK, C), jnp.float32) for _ in range(NRING)],
            pltpu.VMEM_SHARED((NA, C), jnp.float32),
            [pltpu.SemaphoreType.DMA for _ in range(NRING)],
            [pltpu.SemaphoreType.DMA for _ in range(NRING)],
        ],
    )(m, srcp, dst3)


def _mm_kernel(x_ref, w_ref, o_ref):
    o_ref[...] = jnp.dot(x_ref[...], w_ref[...],
                         preferred_element_type=jnp.float32)


def _tc_matmul(x, w):
    return pl.pallas_call(
        _mm_kernel,
        grid=(NBLK,),
        in_specs=[
            pl.BlockSpec((ROWS_BLK, C), lambda i: (i, 0)),
            pl.BlockSpec((C, C), lambda i: (0, 0)),
        ],
        out_specs=pl.BlockSpec((ROWS_BLK, C), lambda i: (i, 0)),
        out_shape=jax.ShapeDtypeStruct((N, C), jnp.float32),
    )(x, w)


def _gru_math(parts_ref, h_ref, wih_ref, whh_ref, bih_ref, bhh_ref):
    agg = parts_ref[0] + parts_ref[1]
    h = h_ref[...]
    gi = jnp.dot(agg, wih_ref[...], preferred_element_type=jnp.float32) \
        + bih_ref[...]
    gh = jnp.dot(h, whh_ref[...], preferred_element_type=jnp.float32) \
        + bhh_ref[...]
    r = jax.nn.sigmoid(gi[:, :C] + gh[:, :C])
    z = jax.nn.sigmoid(gi[:, C:2 * C] + gh[:, C:2 * C])
    n = jnp.tanh(gi[:, 2 * C:] + r * gh[:, 2 * C:])
    return (1.0 - z) * n + z * h


def _gru_next_kernel(parts_ref, h_ref, wih_ref, whh_ref, bih_ref, bhh_ref,
                     wn_ref, h_out, m_out):
    h_new = _gru_math(parts_ref, h_ref, wih_ref, whh_ref, bih_ref, bhh_ref)
    h_out[...] = h_new
    m_out[...] = jnp.dot(h_new, wn_ref[...], preferred_element_type=jnp.float32)


def _gru_last_kernel(parts_ref, h_ref, wih_ref, whh_ref, bih_ref, bhh_ref,
                     h_out):
    h_out[...] = _gru_math(parts_ref, h_ref, wih_ref, whh_ref, bih_ref,
                           bhh_ref)


def _tc_gru(parts, h, wih_t, whh_t, bih2, bhh2, w_next):
    blk = pl.BlockSpec((ROWS_BLK, C), lambda i: (i, 0))
    in_specs = [
        pl.BlockSpec((NC, ROWS_BLK, C), lambda i: (0, i, 0)),
        blk,
        pl.BlockSpec((C, 3 * C), lambda i: (0, 0)),
        pl.BlockSpec((C, 3 * C), lambda i: (0, 0)),
        pl.BlockSpec((1, 3 * C), lambda i: (0, 0)),
        pl.BlockSpec((1, 3 * C), lambda i: (0, 0)),
    ]
    if w_next is not None:
        return pl.pallas_call(
            _gru_next_kernel,
            grid=(NBLK,),
            in_specs=in_specs + [pl.BlockSpec((C, C), lambda i: (0, 0))],
            out_specs=[blk, blk],
            out_shape=[jax.ShapeDtypeStruct((N, C), jnp.float32),
                       jax.ShapeDtypeStruct((N, C), jnp.float32)],
        )(parts, h, wih_t, whh_t, bih2, bhh2, w_next)
    return pl.pallas_call(
        _gru_last_kernel,
        grid=(NBLK,),
        in_specs=in_specs,
        out_specs=blk,
        out_shape=jax.ShapeDtypeStruct((N, C), jnp.float32),
    )(parts, h, wih_t, whh_t, bih2, bhh2)


def _readout_kernel(h_ref, x_ref, b_ref, w1_ref, b1_ref, w2_ref, b2_ref,
                    wo_ref, bo_ref, o_ref, g_acc):
    i = pl.program_id(0)

    @pl.when(i == 0)
    def _():
        g_acc[...] = jnp.zeros_like(g_acc)

    f1 = jax.nn.relu(
        jnp.dot(h_ref[...], w1_ref[:C], preferred_element_type=jnp.float32)
        + jnp.dot(x_ref[...], w1_ref[C:], preferred_element_type=jnp.float32)
        + b1_ref[...])
    f2 = jax.nn.relu(
        jnp.dot(f1, w2_ref[...], preferred_element_type=jnp.float32)
        + b2_ref[...])
    bids = b_ref[0, 0, :]
    seg = lax.broadcasted_iota(jnp.int32, (ROWS_BLK, NUM_GRAPHS), 1)
    onehot = (bids[:, None] == seg).astype(jnp.float32)
    g_acc[...] += lax.dot_general(onehot, f2, (((0,), (0,)), ((), ())),
                                  preferred_element_type=jnp.float32)

    @pl.when(i == NBLK - 1)
    def _():
        o_ref[...] = jax.nn.sigmoid(
            jnp.dot(g_acc[...], wo_ref[...],
                    preferred_element_type=jnp.float32) + bo_ref[...])


def _tc_readout(h, x, batch3, w1_t, b1_2, w2_t, b2_2, wo_t, bo_2):
    blk = pl.BlockSpec((ROWS_BLK, C), lambda i: (i, 0))
    return pl.pallas_call(
        _readout_kernel,
        grid=(NBLK,),
        in_specs=[
            blk,
            blk,
            pl.BlockSpec((1, 1, ROWS_BLK), lambda i: (i, 0, 0)),
            pl.BlockSpec((2 * C, 256), lambda i: (0, 0)),
            pl.BlockSpec((1, 256), lambda i: (0, 0)),
            pl.BlockSpec((256, NUM_GRAPHS), lambda i: (0, 0)),
            pl.BlockSpec((1, NUM_GRAPHS), lambda i: (0, 0)),
            pl.BlockSpec((NUM_GRAPHS, 1), lambda i: (0, 0)),
            pl.BlockSpec((1, 1), lambda i: (0, 0)),
        ],
        out_specs=pl.BlockSpec((NUM_GRAPHS, 1), lambda i: (0, 0)),
        out_shape=jax.ShapeDtypeStruct((NUM_GRAPHS, 1), jnp.float32),
        scratch_shapes=[pltpu.VMEM((NUM_GRAPHS, NUM_GRAPHS), jnp.float32)],
    )(h, x, batch3, w1_t, b1_2, w2_t, b2_2, wo_t, bo_2)


def kernel(x, edge_index, batch, W_ggc, w_ih, w_hh, b_ih, b_hh,
           W1, b1, W2, b2, Wout, bout):
    src = edge_index[0].astype(jnp.int32).reshape(NW, EPW)
    dst = edge_index[1].astype(jnp.int32).reshape(NW, EPW)
    pad = EPWP - EPW
    srcp = jnp.concatenate([src, jnp.zeros((NW, pad), jnp.int32)],
                           axis=1).reshape(-1)
    dstp = jnp.concatenate([dst, jnp.full((NW, pad), N, jnp.int32)], axis=1)
    dst3 = dstp.reshape(NW, NCHUNK, CHUNK)
    batch3 = batch.astype(jnp.int32).reshape(NBLK, 1, ROWS_BLK)

    wih_t = w_ih.T
    whh_t = w_hh.T
    bih2 = b_ih.reshape(1, 3 * C)
    bhh2 = b_hh.reshape(1, 3 * C)
    w1_t = W1.T
    b1_2 = b1.reshape(1, 256)
    w2_t = W2.T
    b2_2 = b2.reshape(1, NUM_GRAPHS)
    wo_t = Wout.T
    bo_2 = bout.reshape(1, 1)

    h = x
    m = _tc_matmul(x, W_ggc[0])
    for i in range(NUM_LAYERS):
        parts = _sc_scatter(m, srcp, dst3).reshape(NC, N, C)
        if i + 1 < NUM_LAYERS:
            h, m = _tc_gru(parts, h, wih_t, whh_t, bih2, bhh2, W_ggc[i + 1])
        else:
            h = _tc_gru(parts, h, wih_t, whh_t, bih2, bhh2, None)
    out = _tc_readout(h, x, batch3, w1_t, b1_2, w2_t, b2_2, wo_t, bo_2)
    return out.reshape(NUM_GRAPHS)


# speedup vs baseline: 1.0326x; 1.0326x over previous
import functools

import jax
import jax.numpy as jnp
from jax import lax
from jax.experimental import pallas as pl
from jax.experimental.pallas import tpu as pltpu
from jax.experimental.pallas import tpu_sc as plsc

N = 10000
E = 320000
C = 128
NUM_LAYERS = 3
NUM_GRAPHS = 64

NC = 2
NS = 16
NW = NC * NS
EPW = E // NW
CHUNK = 96
NCHUNK = -(-EPW // CHUNK)
EPWP = NCHUNK * CHUNK
NRING = 2
NQUAD = NCHUNK // NRING
NA = N + 8
RPT = 624
ZROWS = 16

ROWS_BLK = 1000
NBLK = N // ROWS_BLK


def _sc_scatter_kernel(m_hbm, src_hbm, dst_hbm, out_hbm,
                       sidx, didx, rows, acc, gsems, ssems):
    c = lax.axis_index("c")
    s = lax.axis_index("s")
    wid = c * NS + s

    zero16 = jnp.zeros((16,), jnp.float32)

    def zero_row(r, carry):
        for lane in range(C // 16):
            rows[0][r, pl.ds(lane * 16, 16)] = zero16
        return carry

    lax.fori_loop(0, ZROWS, zero_row, 0)
    zsrc = rows[0].at[pl.ds(0, ZROWS)]

    def zero_acc(t, carry):
        pltpu.sync_copy(zsrc, acc.at[pl.ds(s * RPT + t * ZROWS, ZROWS)])
        return carry

    lax.fori_loop(0, RPT // ZROWS, zero_acc, 0)

    @pl.when(s == NS - 1)
    def _():
        pltpu.sync_copy(zsrc, acc.at[pl.ds(NS * RPT, ZROWS)])
        pltpu.sync_copy(rows[0].at[pl.ds(0, NA - NS * RPT - ZROWS)],
                        acc.at[pl.ds(NS * RPT + ZROWS, NA - NS * RPT - ZROWS)])

    plsc.subcore_barrier()

    pltpu.sync_copy(src_hbm.at[pl.ds(wid * EPWP, EPWP)], sidx)
    pltpu.sync_copy(dst_hbm.at[wid], didx)

    def start_gather(j, b):
        pltpu.async_copy(m_hbm.at[sidx.at[pl.ds(j * CHUNK, CHUNK)]],
                         rows[b], gsems[b])

    def wait_gather(b):
        pltpu.make_async_copy(m_hbm.at[pl.ds(0, CHUNK)],
                              rows[b], gsems[b]).wait()

    for b in range(NRING):
        start_gather(b, b)

    def ring_round(jq, carry):
        j0 = jq * NRING
        descs = []
        for b in range(NRING):
            wait_gather(b)
            descs.append(pltpu.async_copy(rows[b], acc.at[didx.at[j0 + b]],
                                          ssems[b], add=True))
        for b in range(NRING):
            j4 = j0 + NRING + b
            descs[b].wait()

            @pl.when(j4 < NCHUNK)
            def _():
                start_gather(j4, b)
        return carry

    lax.fori_loop(0, NQUAD, ring_round, 0)
    for b in range(NCHUNK - NQUAD * NRING):
        wait_gather(b)
        pltpu.sync_copy(rows[b], acc.at[didx.at[NQUAD * NRING + b]], add=True)
    plsc.subcore_barrier()

    pltpu.sync_copy(acc.at[pl.ds(s * RPT, RPT)],
                    out_hbm.at[pl.ds(c * N + s * RPT, RPT)])

    @pl.when(s == NS - 1)
    def _():
        pltpu.sync_copy(acc.at[pl.ds(NS * RPT, N - NS * RPT)],
                        out_hbm.at[pl.ds(c * N + NS * RPT, N - NS * RPT)])


def _sc_scatter(m, srcp, dst3):
    mesh = plsc.VectorSubcoreMesh(core_axis_name="c", subcore_axis_name="s")
    return pl.kernel(
        _sc_scatter_kernel,
        out_type=jax.ShapeDtypeStruct((NC * N, C), jnp.float32),
        mesh=mesh,
        scratch_types=[
            pltpu.VMEM((EPWP,), jnp.int32),
            pltpu.VMEM((NCHUNK, CHUNK), jnp.int32),
            [pltpu.VMEM((CHUNK, C), jnp.float32) for _ in range(NRING)],
            pltpu.VMEM_SHARED((NA, C), jnp.float32),
            [pltpu.SemaphoreType.DMA for _ in range(NRING)],
            [pltpu.SemaphoreType.DMA for _ in range(NRING)],
        ],
    )(m, srcp, dst3)


def _mm_kernel(x_ref, w_ref, o_ref):
    o_ref[...] = jnp.dot(x_ref[...], w_ref[...],
                         preferred_element_type=jnp.float32)


def _tc_matmul(x, w):
    return pl.pallas_call(
        _mm_kernel,
        grid=(NBLK,),
        in_specs=[
            pl.BlockSpec((ROWS_BLK, C), lambda i: (i, 0)),
            pl.BlockSpec((C, C), lambda i: (0, 0)),
        ],
        out_specs=pl.BlockSpec((ROWS_BLK, C), lambda i: (i, 0)),
        out_shape=jax.ShapeDtypeStruct((N, C), jnp.float32),
    )(x, w)


def _gru_math(parts_ref, h_ref, wih_ref, whh_ref, bih_ref, bhh_ref):
    agg = parts_ref[0] + parts_ref[1]
    h = h_ref[...]
    gi = jnp.dot(agg, wih_ref[...], preferred_element_type=jnp.float32) \
        + bih_ref[...]
    gh = jnp.dot(h, whh_ref[...], preferred_element_type=jnp.float32) \
        + bhh_ref[...]
    r = jax.nn.sigmoid(gi[:, :C] + gh[:, :C])
    z = jax.nn.sigmoid(gi[:, C:2 * C] + gh[:, C:2 * C])
    n = jnp.tanh(gi[:, 2 * C:] + r * gh[:, 2 * C:])
    return (1.0 - z) * n + z * h


def _gru_next_kernel(parts_ref, h_ref, wih_ref, whh_ref, bih_ref, bhh_ref,
                     wn_ref, h_out, m_out):
    h_new = _gru_math(parts_ref, h_ref, wih_ref, whh_ref, bih_ref, bhh_ref)
    h_out[...] = h_new
    m_out[...] = jnp.dot(h_new, wn_ref[...], preferred_element_type=jnp.float32)


def _gru_last_kernel(parts_ref, h_ref, wih_ref, whh_ref, bih_ref, bhh_ref,
                     h_out):
    h_out[...] = _gru_math(parts_ref, h_ref, wih_ref, whh_ref, bih_ref,
                           bhh_ref)


def _tc_gru(parts, h, wih_t, whh_t, bih2, bhh2, w_next):
    blk = pl.BlockSpec((ROWS_BLK, C), lambda i: (i, 0))
    in_specs = [
        pl.BlockSpec((NC, ROWS_BLK, C), lambda i: (0, i, 0)),
        blk,
        pl.BlockSpec((C, 3 * C), lambda i: (0, 0)),
        pl.BlockSpec((C, 3 * C), lambda i: (0, 0)),
        pl.BlockSpec((1, 3 * C), lambda i: (0, 0)),
        pl.BlockSpec((1, 3 * C), lambda i: (0, 0)),
    ]
    if w_next is not None:
        return pl.pallas_call(
            _gru_next_kernel,
            grid=(NBLK,),
            in_specs=in_specs + [pl.BlockSpec((C, C), lambda i: (0, 0))],
            out_specs=[blk, blk],
            out_shape=[jax.ShapeDtypeStruct((N, C), jnp.float32),
                       jax.ShapeDtypeStruct((N, C), jnp.float32)],
        )(parts, h, wih_t, whh_t, bih2, bhh2, w_next)
    return pl.pallas_call(
        _gru_last_kernel,
        grid=(NBLK,),
        in_specs=in_specs,
        out_specs=blk,
        out_shape=jax.ShapeDtypeStruct((N, C), jnp.float32),
    )(parts, h, wih_t, whh_t, bih2, bhh2)


def _readout_kernel(h_ref, x_ref, b_ref, w1_ref, b1_ref, w2_ref, b2_ref,
                    wo_ref, bo_ref, o_ref, g_acc):
    i = pl.program_id(0)

    @pl.when(i == 0)
    def _():
        g_acc[...] = jnp.zeros_like(g_acc)

    f1 = jax.nn.relu(
        jnp.dot(h_ref[...], w1_ref[:C], preferred_element_type=jnp.float32)
        + jnp.dot(x_ref[...], w1_ref[C:], preferred_element_type=jnp.float32)
        + b1_ref[...])
    f2 = jax.nn.relu(
        jnp.dot(f1, w2_ref[...], preferred_element_type=jnp.float32)
        + b2_ref[...])
    bids = b_ref[0, 0, :]
    seg = lax.broadcasted_iota(jnp.int32, (ROWS_BLK, NUM_GRAPHS), 1)
    onehot = (bids[:, None] == seg).astype(jnp.float32)
    g_acc[...] += lax.dot_general(onehot, f2, (((0,), (0,)), ((), ())),
                                  preferred_element_type=jnp.float32)

    @pl.when(i == NBLK - 1)
    def _():
        o_ref[...] = jax.nn.sigmoid(
            jnp.dot(g_acc[...], wo_ref[...],
                    preferred_element_type=jnp.float32) + bo_ref[...])


def _tc_readout(h, x, batch3, w1_t, b1_2, w2_t, b2_2, wo_t, bo_2):
    blk = pl.BlockSpec((ROWS_BLK, C), lambda i: (i, 0))
    return pl.pallas_call(
        _readout_kernel,
        grid=(NBLK,),
        in_specs=[
            blk,
            blk,
            pl.BlockSpec((1, 1, ROWS_BLK), lambda i: (i, 0, 0)),
            pl.BlockSpec((2 * C, 256), lambda i: (0, 0)),
            pl.BlockSpec((1, 256), lambda i: (0, 0)),
            pl.BlockSpec((256, NUM_GRAPHS), lambda i: (0, 0)),
            pl.BlockSpec((1, NUM_GRAPHS), lambda i: (0, 0)),
            pl.BlockSpec((NUM_GRAPHS, 1), lambda i: (0, 0)),
            pl.BlockSpec((1, 1), lambda i: (0, 0)),
        ],
        out_specs=pl.BlockSpec((NUM_GRAPHS, 1), lambda i: (0, 0)),
        out_shape=jax.ShapeDtypeStruct((NUM_GRAPHS, 1), jnp.float32),
        scratch_shapes=[pltpu.VMEM((NUM_GRAPHS, NUM_GRAPHS), jnp.float32)],
    )(h, x, batch3, w1_t, b1_2, w2_t, b2_2, wo_t, bo_2)


def kernel(x, edge_index, batch, W_ggc, w_ih, w_hh, b_ih, b_hh,
           W1, b1, W2, b2, Wout, bout):
    src = edge_index[0].astype(jnp.int32).reshape(NW, EPW)
    dst = edge_index[1].astype(jnp.int32).reshape(NW, EPW)
    pad = EPWP - EPW
    srcp = jnp.concatenate([src, jnp.zeros((NW, pad), jnp.int32)],
                           axis=1).reshape(-1)
    dstp = jnp.concatenate([dst, jnp.full((NW, pad), N, jnp.int32)], axis=1)
    dst3 = dstp.reshape(NW, NCHUNK, CHUNK)
    batch3 = batch.astype(jnp.int32).reshape(NBLK, 1, ROWS_BLK)

    wih_t = w_ih.T
    whh_t = w_hh.T
    bih2 = b_ih.reshape(1, 3 * C)
    bhh2 = b_hh.reshape(1, 3 * C)
    w1_t = W1.T
    b1_2 = b1.reshape(1, 256)
    w2_t = W2.T
    b2_2 = b2.reshape(1, NUM_GRAPHS)
    wo_t = Wout.T
    bo_2 = bout.reshape(1, 1)

    h = x
    m = _tc_matmul(x, W_ggc[0])
    for i in range(NUM_LAYERS):
        parts = _sc_scatter(m, srcp, dst3).reshape(NC, N, C)
        if i + 1 < NUM_LAYERS:
            h, m = _tc_gru(parts, h, wih_t, whh_t, bih2, bhh2, W_ggc[i + 1])
        else:
            h = _tc_gru(parts, h, wih_t, whh_t, bih2, bhh2, None)
    out = _tc_readout(h, x, batch3, w1_t, b1_2, w2_t, b2_2, wo_t, bo_2)
    return out.reshape(NUM_GRAPHS)


# speedup vs baseline: 1.1750x; 1.1379x over previous
import functools

import jax
import jax.numpy as jnp
from jax import lax
from jax.experimental import pallas as pl
from jax.experimental.pallas import tpu as pltpu
from jax.experimental.pallas import tpu_sc as plsc

N = 10000
E = 320000
C = 128
NUM_LAYERS = 3
NUM_GRAPHS = 64

NC = 2
NS = 16
NW = NC * NS
EPW = E // NW
CHUNK = 96
NCHUNK = -(-EPW // CHUNK)
EPWP = NCHUNK * CHUNK
NPAIR = (NCHUNK - 1) // 2
NA = N + 8
RPT = 624
ZROWS = 16

ROWS_BLK = 1000
NBLK = N // ROWS_BLK


def _sc_scatter_kernel(m_hbm, src_hbm, dst_hbm, out_hbm,
                       sidx, didx, rows, acc, gsems, ssems):
    c = lax.axis_index("c")
    s = lax.axis_index("s")
    wid = c * NS + s

    zero16 = jnp.zeros((16,), jnp.float32)

    def zero_row(r, carry):
        for lane in range(C // 16):
            rows[0][r, pl.ds(lane * 16, 16)] = zero16
        return carry

    lax.fori_loop(0, ZROWS, zero_row, 0)
    zsrc = rows[0].at[pl.ds(0, ZROWS)]

    def zero_acc(t, carry):
        pltpu.sync_copy(zsrc, acc.at[pl.ds(s * RPT + t * ZROWS, ZROWS)])
        return carry

    lax.fori_loop(0, RPT // ZROWS, zero_acc, 0)

    @pl.when(s == NS - 1)
    def _():
        pltpu.sync_copy(zsrc, acc.at[pl.ds(NS * RPT, ZROWS)])
        pltpu.sync_copy(rows[0].at[pl.ds(0, NA - NS * RPT - ZROWS)],
                        acc.at[pl.ds(NS * RPT + ZROWS, NA - NS * RPT - ZROWS)])

    plsc.subcore_barrier()

    pltpu.sync_copy(src_hbm.at[pl.ds(wid * EPWP, EPWP)], sidx)
    pltpu.sync_copy(dst_hbm.at[wid], didx)

    def start_gather(j, b):
        pltpu.async_copy(m_hbm.at[sidx.at[pl.ds(j * CHUNK, CHUNK)]],
                         rows[b], gsems[b])

    def wait_gather(b):
        pltpu.make_async_copy(m_hbm.at[pl.ds(0, CHUNK)],
                              rows[b], gsems[b]).wait()

    def scatter(j, b):
        pltpu.sync_copy(rows[b], acc.at[didx.at[j]], add=True)

    start_gather(0, 0)

    def chunk_pair(j2, carry):
        j0 = 2 * j2
        start_gather(j0 + 1, 1)
        wait_gather(0)
        scatter(j0, 0)
        start_gather(j0 + 2, 0)
        wait_gather(1)
        scatter(j0 + 1, 1)
        return carry

    lax.fori_loop(0, NPAIR, chunk_pair, 0)
    wait_gather(0)
    scatter(NCHUNK - 1, 0)
    plsc.subcore_barrier()

    pltpu.sync_copy(acc.at[pl.ds(s * RPT, RPT)],
                    out_hbm.at[pl.ds(c * N + s * RPT, RPT)])

    @pl.when(s == NS - 1)
    def _():
        pltpu.sync_copy(acc.at[pl.ds(NS * RPT, N - NS * RPT)],
                        out_hbm.at[pl.ds(c * N + NS * RPT, N - NS * RPT)])


def _sc_scatter(m, srcp, dst3):
    mesh = plsc.VectorSubcoreMesh(core_axis_name="c", subcore_axis_name="s")
    return pl.kernel(
        _sc_scatter_kernel,
        out_type=jax.ShapeDtypeStruct((NC * N, C), jnp.float32),
        mesh=mesh,
        scratch_types=[
            pltpu.VMEM((EPWP,), jnp.int32),
            pltpu.VMEM((NCHUNK, CHUNK), jnp.int32),
            [pltpu.VMEM((CHUNK, C), jnp.float32) for _ in range(2)],
            pltpu.VMEM_SHARED((NA, C), jnp.float32),
            [pltpu.SemaphoreType.DMA for _ in range(2)],
            [pltpu.SemaphoreType.DMA for _ in range(2)],
        ],
    )(m, srcp, dst3)


def _mm_kernel(x_ref, w_ref, o_ref):
    o_ref[...] = jnp.dot(x_ref[...], w_ref[...],
                         preferred_element_type=jnp.float32)


def _tc_matmul(x, w):
    return pl.pallas_call(
        _mm_kernel,
        grid=(NBLK,),
        in_specs=[
            pl.BlockSpec((ROWS_BLK, C), lambda i: (i, 0)),
            pl.BlockSpec((C, C), lambda i: (0, 0)),
        ],
        out_specs=pl.BlockSpec((ROWS_BLK, C), lambda i: (i, 0)),
        out_shape=jax.ShapeDtypeStruct((N, C), jnp.float32),
    )(x, w)


def _gru_math(parts_ref, h_ref, wih_ref, whh_ref, bih_ref, bhh_ref):
    agg = parts_ref[0] + parts_ref[1]
    h = h_ref[...]
    gi = jnp.dot(agg, wih_ref[...], preferred_element_type=jnp.float32) \
        + bih_ref[...]
    gh = jnp.dot(h, whh_ref[...], preferred_element_type=jnp.float32) \
        + bhh_ref[...]
    r = jax.nn.sigmoid(gi[:, :C] + gh[:, :C])
    z = jax.nn.sigmoid(gi[:, C:2 * C] + gh[:, C:2 * C])
    n = jnp.tanh(gi[:, 2 * C:] + r * gh[:, 2 * C:])
    return (1.0 - z) * n + z * h


def _gru_next_kernel(parts_ref, h_ref, wih_ref, whh_ref, bih_ref, bhh_ref,
                     wn_ref, h_out, m_out):
    h_new = _gru_math(parts_ref, h_ref, wih_ref, whh_ref, bih_ref, bhh_ref)
    h_out[...] = h_new
    m_out[...] = jnp.dot(h_new, wn_ref[...], preferred_element_type=jnp.float32)


def _gru_last_kernel(parts_ref, h_ref, wih_ref, whh_ref, bih_ref, bhh_ref,
                     h_out):
    h_out[...] = _gru_math(parts_ref, h_ref, wih_ref, whh_ref, bih_ref,
                           bhh_ref)


def _tc_gru(parts, h, wih_t, whh_t, bih2, bhh2, w_next):
    blk = pl.BlockSpec((ROWS_BLK, C), lambda i: (i, 0))
    in_specs = [
        pl.BlockSpec((NC, ROWS_BLK, C), lambda i: (0, i, 0)),
        blk,
        pl.BlockSpec((C, 3 * C), lambda i: (0, 0)),
        pl.BlockSpec((C, 3 * C), lambda i: (0, 0)),
        pl.BlockSpec((1, 3 * C), lambda i: (0, 0)),
        pl.BlockSpec((1, 3 * C), lambda i: (0, 0)),
    ]
    if w_next is not None:
        return pl.pallas_call(
            _gru_next_kernel,
            grid=(NBLK,),
            in_specs=in_specs + [pl.BlockSpec((C, C), lambda i: (0, 0))],
            out_specs=[blk, blk],
            out_shape=[jax.ShapeDtypeStruct((N, C), jnp.float32),
                       jax.ShapeDtypeStruct((N, C), jnp.float32)],
        )(parts, h, wih_t, whh_t, bih2, bhh2, w_next)
    return pl.pallas_call(
        _gru_last_kernel,
        grid=(NBLK,),
        in_specs=in_specs,
        out_specs=blk,
        out_shape=jax.ShapeDtypeStruct((N, C), jnp.float32),
    )(parts, h, wih_t, whh_t, bih2, bhh2)


def _readout_kernel(h_ref, x_ref, b_ref, w1_ref, b1_ref, w2_ref, b2_ref,
                    wo_ref, bo_ref, o_ref, g_acc):
    i = pl.program_id(0)

    @pl.when(i == 0)
    def _():
        g_acc[...] = jnp.zeros_like(g_acc)

    f1 = jax.nn.relu(
        jnp.dot(h_ref[...], w1_ref[:C], preferred_element_type=jnp.float32)
        + jnp.dot(x_ref[...], w1_ref[C:], preferred_element_type=jnp.float32)
        + b1_ref[...])
    f2 = jax.nn.relu(
        jnp.dot(f1, w2_ref[...], preferred_element_type=jnp.float32)
        + b2_ref[...])
    bids = b_ref[0, 0, :]
    seg = lax.broadcasted_iota(jnp.int32, (ROWS_BLK, NUM_GRAPHS), 1)
    onehot = (bids[:, None] == seg).astype(jnp.float32)
    g_acc[...] += lax.dot_general(onehot, f2, (((0,), (0,)), ((), ())),
                                  preferred_element_type=jnp.float32)

    @pl.when(i == NBLK - 1)
    def _():
        o_ref[...] = jax.nn.sigmoid(
            jnp.dot(g_acc[...], wo_ref[...],
                    preferred_element_type=jnp.float32) + bo_ref[...])


def _tc_readout(h, x, batch3, w1_t, b1_2, w2_t, b2_2, wo_t, bo_2):
    blk = pl.BlockSpec((ROWS_BLK, C), lambda i: (i, 0))
    return pl.pallas_call(
        _readout_kernel,
        grid=(NBLK,),
        in_specs=[
            blk,
            blk,
            pl.BlockSpec((1, 1, ROWS_BLK), lambda i: (i, 0, 0)),
            pl.BlockSpec((2 * C, 256), lambda i: (0, 0)),
            pl.BlockSpec((1, 256), lambda i: (0, 0)),
            pl.BlockSpec((256, NUM_GRAPHS), lambda i: (0, 0)),
            pl.BlockSpec((1, NUM_GRAPHS), lambda i: (0, 0)),
            pl.BlockSpec((NUM_GRAPHS, 1), lambda i: (0, 0)),
            pl.BlockSpec((1, 1), lambda i: (0, 0)),
        ],
        out_specs=pl.BlockSpec((NUM_GRAPHS, 1), lambda i: (0, 0)),
        out_shape=jax.ShapeDtypeStruct((NUM_GRAPHS, 1), jnp.float32),
        scratch_shapes=[pltpu.VMEM((NUM_GRAPHS, NUM_GRAPHS), jnp.float32)],
    )(h, x, batch3, w1_t, b1_2, w2_t, b2_2, wo_t, bo_2)


def kernel(x, edge_index, batch, W_ggc, w_ih, w_hh, b_ih, b_hh,
           W1, b1, W2, b2, Wout, bout):
    src = edge_index[0].astype(jnp.int32).reshape(NW, EPW)
    dst = edge_index[1].astype(jnp.int32).reshape(NW, EPW)
    pad = EPWP - EPW
    srcp = jnp.concatenate([src, jnp.zeros((NW, pad), jnp.int32)],
                           axis=1).reshape(-1)
    dstp = jnp.concatenate([dst, jnp.full((NW, pad), N, jnp.int32)], axis=1)
    dst3 = dstp.reshape(NW, NCHUNK, CHUNK)
    batch3 = batch.astype(jnp.int32).reshape(NBLK, 1, ROWS_BLK)

    wih_t = w_ih.T
    whh_t = w_hh.T
    bih2 = b_ih.reshape(1, 3 * C)
    bhh2 = b_hh.reshape(1, 3 * C)
    w1_t = W1.T
    b1_2 = b1.reshape(1, 256)
    w2_t = W2.T
    b2_2 = b2.reshape(1, NUM_GRAPHS)
    wo_t = Wout.T
    bo_2 = bout.reshape(1, 1)

    h = x
    m = _tc_matmul(x, W_ggc[0])
    for i in range(NUM_LAYERS):
        parts = _sc_scatter(m, srcp, dst3).reshape(NC, N, C)
        if i + 1 < NUM_LAYERS:
            h, m = _tc_gru(parts, h, wih_t, whh_t, bih2, bhh2, W_ggc[i + 1])
        else:
            h = _tc_gru(parts, h, wih_t, whh_t, bih2, bhh2, None)
    out = _tc_readout(h, x, batch3, w1_t, b1_2, w2_t, b2_2, wo_t, bo_2)
    return out.reshape(NUM_GRAPHS)


# speedup vs baseline: 2.1968x; 1.8697x over previous
import functools

import jax
import jax.numpy as jnp
from jax import lax
from jax.experimental import pallas as pl
from jax.experimental.pallas import tpu as pltpu
from jax.experimental.pallas import tpu_sc as plsc

N = 10000
E = 320000
C = 128
NUM_LAYERS = 3
NUM_GRAPHS = 64

NC = 2
NS = 16
NW = NC * NS
EPW = E // NW
CHUNK = 80
NCHUNK = EPW // CHUNK
EPWP = NCHUNK * CHUNK
NRING = 3
NROUND = -(-NCHUNK // NRING)
NA = N
RPT = 624
ZROWS = 16

ROWS_BLK = 1000
NBLK = N // ROWS_BLK


def _sc_scatter_kernel(m_hbm, src_hbm, dst_hbm, out_hbm,
                       sidx, didxb, rows, acc, gsems, isems):
    c = lax.axis_index("c")
    s = lax.axis_index("s")
    wid = c * NS + s

    zero16 = jnp.zeros((16,), jnp.float32)

    def zero_row(r, carry):
        for lane in range(C // 16):
            rows[0][r, pl.ds(lane * 16, 16)] = zero16
        return carry

    lax.fori_loop(0, ZROWS, zero_row, 0)
    zsrc = rows[0].at[pl.ds(0, ZROWS)]

    def zero_acc(t, carry):
        pltpu.sync_copy(zsrc, acc.at[pl.ds(s * RPT + t * ZROWS, ZROWS)])
        return carry

    lax.fori_loop(0, RPT // ZROWS, zero_acc, 0)

    @pl.when(s == NS - 1)
    def _():
        pltpu.sync_copy(zsrc, acc.at[pl.ds(NS * RPT, ZROWS)])

    plsc.subcore_barrier()

    pltpu.sync_copy(src_hbm.at[pl.ds(wid * EPWP, EPWP)], sidx)

    def start_chunk(j, b):
        pltpu.async_copy(dst_hbm.at[pl.ds(wid * EPWP + j * CHUNK, CHUNK)],
                         didxb[b], isems[b])
        pltpu.async_copy(m_hbm.at[sidx.at[pl.ds(j * CHUNK, CHUNK)]],
                         rows[b], gsems[b])

    def wait_chunk(b):
        pltpu.make_async_copy(dst_hbm.at[pl.ds(0, CHUNK)],
                              didxb[b], isems[b]).wait()
        pltpu.make_async_copy(m_hbm.at[pl.ds(0, CHUNK)],
                              rows[b], gsems[b]).wait()

    for b in range(NRING):
        start_chunk(b, b)

    def ring_round(r, carry):
        j0 = r * NRING
        for b in range(NRING):
            j = j0 + b

            @pl.when(j < NCHUNK)
            def _():
                wait_chunk(b)
                pltpu.sync_copy(rows[b], acc.at[didxb[b]], add=True)

                @pl.when(j + NRING < NCHUNK)
                def _():
                    start_chunk(j + NRING, b)
        return carry

    lax.fori_loop(0, NROUND, ring_round, 0)
    plsc.subcore_barrier()

    pltpu.sync_copy(acc.at[pl.ds(s * RPT, RPT)],
                    out_hbm.at[pl.ds(c * N + s * RPT, RPT)])

    @pl.when(s == NS - 1)
    def _():
        pltpu.sync_copy(acc.at[pl.ds(NS * RPT, N - NS * RPT)],
                        out_hbm.at[pl.ds(c * N + NS * RPT, N - NS * RPT)])


def _sc_scatter(m, srcp, dstp):
    mesh = plsc.VectorSubcoreMesh(core_axis_name="c", subcore_axis_name="s")
    return pl.kernel(
        _sc_scatter_kernel,
        out_type=jax.ShapeDtypeStruct((NC * N, C), jnp.float32),
        mesh=mesh,
        scratch_types=[
            pltpu.VMEM((EPWP,), jnp.int32),
            [pltpu.VMEM((CHUNK,), jnp.int32) for _ in range(NRING)],
            [pltpu.VMEM((CHUNK, C), jnp.float32) for _ in range(NRING)],
            pltpu.VMEM_SHARED((NA, C), jnp.float32),
            [pltpu.SemaphoreType.DMA for _ in range(NRING)],
            [pltpu.SemaphoreType.DMA for _ in range(NRING)],
        ],
    )(m, srcp, dstp)


def _mm_kernel(x_ref, w_ref, o_ref):
    o_ref[...] = jnp.dot(x_ref[...], w_ref[...],
                         preferred_element_type=jnp.float32)


def _tc_matmul(x, w):
    return pl.pallas_call(
        _mm_kernel,
        grid=(NBLK,),
        in_specs=[
            pl.BlockSpec((ROWS_BLK, C), lambda i: (i, 0)),
            pl.BlockSpec((C, C), lambda i: (0, 0)),
        ],
        out_specs=pl.BlockSpec((ROWS_BLK, C), lambda i: (i, 0)),
        out_shape=jax.ShapeDtypeStruct((N, C), jnp.float32),
    )(x, w)


def _gru_math(parts_ref, h_ref, wih_ref, whh_ref, bih_ref, bhh_ref):
    agg = parts_ref[0] + parts_ref[1]
    h = h_ref[...]
    gi = jnp.dot(agg, wih_ref[...], preferred_element_type=jnp.float32) \
        + bih_ref[...]
    gh = jnp.dot(h, whh_ref[...], preferred_element_type=jnp.float32) \
        + bhh_ref[...]
    r = jax.nn.sigmoid(gi[:, :C] + gh[:, :C])
    z = jax.nn.sigmoid(gi[:, C:2 * C] + gh[:, C:2 * C])
    n = jnp.tanh(gi[:, 2 * C:] + r * gh[:, 2 * C:])
    return (1.0 - z) * n + z * h


def _gru_next_kernel(parts_ref, h_ref, wih_ref, whh_ref, bih_ref, bhh_ref,
                     wn_ref, h_out, m_out):
    h_new = _gru_math(parts_ref, h_ref, wih_ref, whh_ref, bih_ref, bhh_ref)
    h_out[...] = h_new
    m_out[...] = jnp.dot(h_new, wn_ref[...], preferred_element_type=jnp.float32)


def _gru_last_kernel(parts_ref, h_ref, wih_ref, whh_ref, bih_ref, bhh_ref,
                     h_out):
    h_out[...] = _gru_math(parts_ref, h_ref, wih_ref, whh_ref, bih_ref,
                           bhh_ref)


def _tc_gru(parts, h, wih_t, whh_t, bih2, bhh2, w_next):
    blk = pl.BlockSpec((ROWS_BLK, C), lambda i: (i, 0))
    in_specs = [
        pl.BlockSpec((NC, ROWS_BLK, C), lambda i: (0, i, 0)),
        blk,
        pl.BlockSpec((C, 3 * C), lambda i: (0, 0)),
        pl.BlockSpec((C, 3 * C), lambda i: (0, 0)),
        pl.BlockSpec((1, 3 * C), lambda i: (0, 0)),
        pl.BlockSpec((1, 3 * C), lambda i: (0, 0)),
    ]
    if w_next is not None:
        return pl.pallas_call(
            _gru_next_kernel,
            grid=(NBLK,),
            in_specs=in_specs + [pl.BlockSpec((C, C), lambda i: (0, 0))],
            out_specs=[blk, blk],
            out_shape=[jax.ShapeDtypeStruct((N, C), jnp.float32),
                       jax.ShapeDtypeStruct((N, C), jnp.float32)],
        )(parts, h, wih_t, whh_t, bih2, bhh2, w_next)
    return pl.pallas_call(
        _gru_last_kernel,
        grid=(NBLK,),
        in_specs=in_specs,
        out_specs=blk,
        out_shape=jax.ShapeDtypeStruct((N, C), jnp.float32),
    )(parts, h, wih_t, whh_t, bih2, bhh2)


def _readout_kernel(h_ref, x_ref, b_ref, w1_ref, b1_ref, w2_ref, b2_ref,
                    wo_ref, bo_ref, o_ref, g_acc):
    i = pl.program_id(0)

    @pl.when(i == 0)
    def _():
        g_acc[...] = jnp.zeros_like(g_acc)

    f1 = jax.nn.relu(
        jnp.dot(h_ref[...], w1_ref[:C], preferred_element_type=jnp.float32)
        + jnp.dot(x_ref[...], w1_ref[C:], preferred_element_type=jnp.float32)
        + b1_ref[...])
    f2 = jax.nn.relu(
        jnp.dot(f1, w2_ref[...], preferred_element_type=jnp.float32)
        + b2_ref[...])
    bids = b_ref[0, 0, :]
    seg = lax.broadcasted_iota(jnp.int32, (ROWS_BLK, NUM_GRAPHS), 1)
    onehot = (bids[:, None] == seg).astype(jnp.float32)
    g_acc[...] += lax.dot_general(onehot, f2, (((0,), (0,)), ((), ())),
                                  preferred_element_type=jnp.float32)

    @pl.when(i == NBLK - 1)
    def _():
        o_ref[...] = jax.nn.sigmoid(
            jnp.dot(g_acc[...], wo_ref[...],
                    preferred_element_type=jnp.float32) + bo_ref[...])


def _tc_readout(h, x, batch3, w1_t, b1_2, w2_t, b2_2, wo_t, bo_2):
    blk = pl.BlockSpec((ROWS_BLK, C), lambda i: (i, 0))
    return pl.pallas_call(
        _readout_kernel,
        grid=(NBLK,),
        in_specs=[
            blk,
            blk,
            pl.BlockSpec((1, 1, ROWS_BLK), lambda i: (i, 0, 0)),
            pl.BlockSpec((2 * C, 256), lambda i: (0, 0)),
            pl.BlockSpec((1, 256), lambda i: (0, 0)),
            pl.BlockSpec((256, NUM_GRAPHS), lambda i: (0, 0)),
            pl.BlockSpec((1, NUM_GRAPHS), lambda i: (0, 0)),
            pl.BlockSpec((NUM_GRAPHS, 1), lambda i: (0, 0)),
            pl.BlockSpec((1, 1), lambda i: (0, 0)),
        ],
        out_specs=pl.BlockSpec((NUM_GRAPHS, 1), lambda i: (0, 0)),
        out_shape=jax.ShapeDtypeStruct((NUM_GRAPHS, 1), jnp.float32),
        scratch_shapes=[pltpu.VMEM((NUM_GRAPHS, NUM_GRAPHS), jnp.float32)],
    )(h, x, batch3, w1_t, b1_2, w2_t, b2_2, wo_t, bo_2)


def kernel(x, edge_index, batch, W_ggc, w_ih, w_hh, b_ih, b_hh,
           W1, b1, W2, b2, Wout, bout):
    srcp = edge_index[0].astype(jnp.int32)
    dstp = edge_index[1].astype(jnp.int32)
    batch3 = batch.astype(jnp.int32).reshape(NBLK, 1, ROWS_BLK)

    wih_t = w_ih.T
    whh_t = w_hh.T
    bih2 = b_ih.reshape(1, 3 * C)
    bhh2 = b_hh.reshape(1, 3 * C)
    w1_t = W1.T
    b1_2 = b1.reshape(1, 256)
    w2_t = W2.T
    b2_2 = b2.reshape(1, NUM_GRAPHS)
    wo_t = Wout.T
    bo_2 = bout.reshape(1, 1)

    h = x
    m = _tc_matmul(x, W_ggc[0])
    for i in range(NUM_LAYERS):
        parts = _sc_scatter(m, srcp, dstp).reshape(NC, N, C)
        if i + 1 < NUM_LAYERS:
            h, m = _tc_gru(parts, h, wih_t, whh_t, bih2, bhh2, W_ggc[i + 1])
        else:
            h = _tc_gru(parts, h, wih_t, whh_t, bih2, bhh2, None)
    out = _tc_readout(h, x, batch3, w1_t, b1_2, w2_t, b2_2, wo_t, bo_2)
    return out.reshape(NUM_GRAPHS)


# speedup vs baseline: 2.2455x; 1.0222x over previous
import functools

import jax
import jax.numpy as jnp
from jax import lax
from jax.experimental import pallas as pl
from jax.experimental.pallas import tpu as pltpu
from jax.experimental.pallas import tpu_sc as plsc

N = 10000
E = 320000
C = 128
NUM_LAYERS = 3
NUM_GRAPHS = 64

NC = 2
NS = 16
NW = NC * NS
EPW = E // NW
CHUNK = 64
NCHUNK = EPW // CHUNK
TAIL = EPW - NCHUNK * CHUNK
NRING = 4
NROUND = NCHUNK // NRING
NA = N
RPT = 624
ZROWS = 16

ROWS_BLK = 1000
NBLK = N // ROWS_BLK


def _sc_scatter_kernel(m_hbm, src_hbm, dst_hbm, out_hbm,
                       sidx, didxb, dtail, rows, acc, gsems, isems):
    c = lax.axis_index("c")
    s = lax.axis_index("s")
    wid = c * NS + s

    zero16 = jnp.zeros((16,), jnp.float32)

    def zero_row(r, carry):
        for lane in range(C // 16):
            rows[0][r, pl.ds(lane * 16, 16)] = zero16
        return carry

    lax.fori_loop(0, ZROWS, zero_row, 0)
    zsrc = rows[0].at[pl.ds(0, ZROWS)]

    def zero_acc(t, carry):
        pltpu.sync_copy(zsrc, acc.at[pl.ds(s * RPT + t * ZROWS, ZROWS)])
        return carry

    lax.fori_loop(0, RPT // ZROWS, zero_acc, 0)

    @pl.when(s == NS - 1)
    def _():
        pltpu.sync_copy(zsrc, acc.at[pl.ds(NS * RPT, ZROWS)])

    plsc.subcore_barrier()

    pltpu.sync_copy(src_hbm.at[pl.ds(wid * EPW, EPW)], sidx)

    def start_chunk(j, b):
        pltpu.async_copy(dst_hbm.at[pl.ds(wid * EPW + j * CHUNK, CHUNK)],
                         didxb[b], isems[b])
        pltpu.async_copy(m_hbm.at[sidx.at[pl.ds(j * CHUNK, CHUNK)]],
                         rows[b], gsems[b])

    def wait_chunk(b):
        pltpu.make_async_copy(dst_hbm.at[pl.ds(0, CHUNK)],
                              didxb[b], isems[b]).wait()
        pltpu.make_async_copy(m_hbm.at[pl.ds(0, CHUNK)],
                              rows[b], gsems[b]).wait()

    for b in range(NRING):
        start_chunk(b, b)

    def ring_round(r, carry):
        j0 = r * NRING
        for b in range(NRING):
            j = j0 + b
            wait_chunk(b)
            pltpu.sync_copy(rows[b], acc.at[didxb[b]], add=True)

            @pl.when(j + NRING < NCHUNK)
            def _():
                start_chunk(j + NRING, b)
        return carry

    lax.fori_loop(0, NROUND, ring_round, 0)

    tbase = wid * EPW + NCHUNK * CHUNK
    pltpu.sync_copy(dst_hbm.at[pl.ds(tbase, TAIL)], dtail)
    rtail = rows[0].at[pl.ds(0, TAIL)]
    pltpu.async_copy(m_hbm.at[sidx.at[pl.ds(NCHUNK * CHUNK, TAIL)]],
                     rtail, gsems[0])
    pltpu.make_async_copy(m_hbm.at[pl.ds(0, TAIL)], rtail, gsems[0]).wait()
    pltpu.sync_copy(rtail, acc.at[dtail], add=True)
    plsc.subcore_barrier()

    pltpu.sync_copy(acc.at[pl.ds(s * RPT, RPT)],
                    out_hbm.at[pl.ds(c * N + s * RPT, RPT)])

    @pl.when(s == NS - 1)
    def _():
        pltpu.sync_copy(acc.at[pl.ds(NS * RPT, N - NS * RPT)],
                        out_hbm.at[pl.ds(c * N + NS * RPT, N - NS * RPT)])


def _sc_scatter(m, srcp, dstp):
    mesh = plsc.VectorSubcoreMesh(core_axis_name="c", subcore_axis_name="s")
    return pl.kernel(
        _sc_scatter_kernel,
        out_type=jax.ShapeDtypeStruct((NC * N, C), jnp.float32),
        mesh=mesh,
        scratch_types=[
            pltpu.VMEM((EPW,), jnp.int32),
            [pltpu.VMEM((CHUNK,), jnp.int32) for _ in range(NRING)],
            pltpu.VMEM((TAIL,), jnp.int32),
            [pltpu.VMEM((CHUNK, C), jnp.float32) for _ in range(NRING)],
            pltpu.VMEM_SHARED((NA, C), jnp.float32),
            [pltpu.SemaphoreType.DMA for _ in range(NRING)],
            [pltpu.SemaphoreType.DMA for _ in range(NRING)],
        ],
    )(m, srcp, dstp)


def _mm_kernel(x_ref, w_ref, o_ref):
    o_ref[...] = jnp.dot(x_ref[...], w_ref[...],
                         preferred_element_type=jnp.float32)


def _tc_matmul(x, w):
    return pl.pallas_call(
        _mm_kernel,
        grid=(NBLK,),
        in_specs=[
            pl.BlockSpec((ROWS_BLK, C), lambda i: (i, 0)),
            pl.BlockSpec((C, C), lambda i: (0, 0)),
        ],
        out_specs=pl.BlockSpec((ROWS_BLK, C), lambda i: (i, 0)),
        out_shape=jax.ShapeDtypeStruct((N, C), jnp.float32),
    )(x, w)


def _gru_math(parts_ref, h_ref, wih_ref, whh_ref, bih_ref, bhh_ref):
    agg = parts_ref[0] + parts_ref[1]
    h = h_ref[...]
    gi = jnp.dot(agg, wih_ref[...], preferred_element_type=jnp.float32) \
        + bih_ref[...]
    gh = jnp.dot(h, whh_ref[...], preferred_element_type=jnp.float32) \
        + bhh_ref[...]
    r = jax.nn.sigmoid(gi[:, :C] + gh[:, :C])
    z = jax.nn.sigmoid(gi[:, C:2 * C] + gh[:, C:2 * C])
    n = jnp.tanh(gi[:, 2 * C:] + r * gh[:, 2 * C:])
    return (1.0 - z) * n + z * h


def _gru_next_kernel(parts_ref, h_ref, wih_ref, whh_ref, bih_ref, bhh_ref,
                     wn_ref, h_out, m_out):
    h_new = _gru_math(parts_ref, h_ref, wih_ref, whh_ref, bih_ref, bhh_ref)
    h_out[...] = h_new
    m_out[...] = jnp.dot(h_new, wn_ref[...], preferred_element_type=jnp.float32)


def _gru_last_kernel(parts_ref, h_ref, wih_ref, whh_ref, bih_ref, bhh_ref,
                     h_out):
    h_out[...] = _gru_math(parts_ref, h_ref, wih_ref, whh_ref, bih_ref,
                           bhh_ref)


def _tc_gru(parts, h, wih_t, whh_t, bih2, bhh2, w_next):
    blk = pl.BlockSpec((ROWS_BLK, C), lambda i: (i, 0))
    in_specs = [
        pl.BlockSpec((NC, ROWS_BLK, C), lambda i: (0, i, 0)),
        blk,
        pl.BlockSpec((C, 3 * C), lambda i: (0, 0)),
        pl.BlockSpec((C, 3 * C), lambda i: (0, 0)),
        pl.BlockSpec((1, 3 * C), lambda i: (0, 0)),
        pl.BlockSpec((1, 3 * C), lambda i: (0, 0)),
    ]
    if w_next is not None:
        return pl.pallas_call(
            _gru_next_kernel,
            grid=(NBLK,),
            in_specs=in_specs + [pl.BlockSpec((C, C), lambda i: (0, 0))],
            out_specs=[blk, blk],
            out_shape=[jax.ShapeDtypeStruct((N, C), jnp.float32),
                       jax.ShapeDtypeStruct((N, C), jnp.float32)],
        )(parts, h, wih_t, whh_t, bih2, bhh2, w_next)
    return pl.pallas_call(
        _gru_last_kernel,
        grid=(NBLK,),
        in_specs=in_specs,
        out_specs=blk,
        out_shape=jax.ShapeDtypeStruct((N, C), jnp.float32),
    )(parts, h, wih_t, whh_t, bih2, bhh2)


def _readout_kernel(h_ref, x_ref, b_ref, w1_ref, b1_ref, w2_ref, b2_ref,
                    wo_ref, bo_ref, o_ref, g_acc):
    i = pl.program_id(0)

    @pl.when(i == 0)
    def _():
        g_acc[...] = jnp.zeros_like(g_acc)

    f1 = jax.nn.relu(
        jnp.dot(h_ref[...], w1_ref[:C], preferred_element_type=jnp.float32)
        + jnp.dot(x_ref[...], w1_ref[C:], preferred_element_type=jnp.float32)
        + b1_ref[...])
    f2 = jax.nn.relu(
        jnp.dot(f1, w2_ref[...], preferred_element_type=jnp.float32)
        + b2_ref[...])
    bids = b_ref[0, 0, :]
    seg = lax.broadcasted_iota(jnp.int32, (ROWS_BLK, NUM_GRAPHS), 1)
    onehot = (bids[:, None] == seg).astype(jnp.float32)
    g_acc[...] += lax.dot_general(onehot, f2, (((0,), (0,)), ((), ())),
                                  preferred_element_type=jnp.float32)

    @pl.when(i == NBLK - 1)
    def _():
        o_ref[...] = jax.nn.sigmoid(
            jnp.dot(g_acc[...], wo_ref[...],
                    preferred_element_type=jnp.float32) + bo_ref[...])


def _tc_readout(h, x, batch3, w1_t, b1_2, w2_t, b2_2, wo_t, bo_2):
    blk = pl.BlockSpec((ROWS_BLK, C), lambda i: (i, 0))
    return pl.pallas_call(
        _readout_kernel,
        grid=(NBLK,),
        in_specs=[
            blk,
            blk,
            pl.BlockSpec((1, 1, ROWS_BLK), lambda i: (i, 0, 0)),
            pl.BlockSpec((2 * C, 256), lambda i: (0, 0)),
            pl.BlockSpec((1, 256), lambda i: (0, 0)),
            pl.BlockSpec((256, NUM_GRAPHS), lambda i: (0, 0)),
            pl.BlockSpec((1, NUM_GRAPHS), lambda i: (0, 0)),
            pl.BlockSpec((NUM_GRAPHS, 1), lambda i: (0, 0)),
            pl.BlockSpec((1, 1), lambda i: (0, 0)),
        ],
        out_specs=pl.BlockSpec((NUM_GRAPHS, 1), lambda i: (0, 0)),
        out_shape=jax.ShapeDtypeStruct((NUM_GRAPHS, 1), jnp.float32),
        scratch_shapes=[pltpu.VMEM((NUM_GRAPHS, NUM_GRAPHS), jnp.float32)],
    )(h, x, batch3, w1_t, b1_2, w2_t, b2_2, wo_t, bo_2)


def kernel(x, edge_index, batch, W_ggc, w_ih, w_hh, b_ih, b_hh,
           W1, b1, W2, b2, Wout, bout):
    srcp = edge_index[0].astype(jnp.int32)
    dstp = edge_index[1].astype(jnp.int32)
    batch3 = batch.astype(jnp.int32).reshape(NBLK, 1, ROWS_BLK)

    wih_t = w_ih.T
    whh_t = w_hh.T
    bih2 = b_ih.reshape(1, 3 * C)
    bhh2 = b_hh.reshape(1, 3 * C)
    w1_t = W1.T
    b1_2 = b1.reshape(1, 256)
    w2_t = W2.T
    b2_2 = b2.reshape(1, NUM_GRAPHS)
    wo_t = Wout.T
    bo_2 = bout.reshape(1, 1)

    h = x
    m = _tc_matmul(x, W_ggc[0])
    for i in range(NUM_LAYERS):
        parts = _sc_scatter(m, srcp, dstp).reshape(NC, N, C)
        if i + 1 < NUM_LAYERS:
            h, m = _tc_gru(parts, h, wih_t, whh_t, bih2, bhh2, W_ggc[i + 1])
        else:
            h = _tc_gru(parts, h, wih_t, whh_t, bih2, bhh2, None)
    out = _tc_readout(h, x, batch3, w1_t, b1_2, w2_t, b2_2, wo_t, bo_2)
    return out.reshape(NUM_GRAPHS)
